# baseline (speedup 1.0000x reference)
import functools

import jax
import jax.numpy as jnp
from jax import lax
from jax.experimental import pallas as pl
from jax.experimental.pallas import tpu as pltpu
from jax.experimental.pallas import tpu_sc as plsc

S = 12
SS = S * S
C = 96
H = 512
W = 512
NW = 32
RPW = 32
RPAD = NW * RPW
NCHUNK = SS // 16


def _splat(vec, lane):
    idx = jnp.full((16, 1), lane, jnp.int32)
    dnums = lax.GatherDimensionNumbers(
        offset_dims=(), collapsed_slice_dims=(0,), start_index_map=(0,))
    return lax.gather(vec, idx, dnums, (1,),
                      mode=lax.GatherScatterMode.PROMISE_IN_BOUNDS)


def _sc_body(xt_hbm, rois_hbm, out_hbm, roisv, idxv, wbuf, gbuf, obuf, sem):
    wid = lax.axis_index("s") * 2 + lax.axis_index("c")
    rbase = wid * RPW
    pltpu.sync_copy(rois_hbm.at[pl.ds(rbase, RPW)], roisv)

    lanes = lax.iota(jnp.int32, 16)

    def roi_body(i, carry):
        row = roisv[i, :]
        x1v = _splat(row, 1)
        y1v = _splat(row, 2)
        x2v = _splat(row, 3)
        y2v = _splat(row, 4)
        bwv = (x2v - x1v) / float(S)
        bhv = (y2v - y1v) / float(S)

        def chunk_body(j, carry2):
            s_i = lanes + j * 16
            sy = lax.div(s_i, S)
            sx = s_i - sy * S
            Yv = y1v + (sy.astype(jnp.float32) + 0.5) * bhv
            Xv = x1v + (sx.astype(jnp.float32) + 0.5) * bwv
            y0i = Yv.astype(jnp.int32)
            x0i = Xv.astype(jnp.int32)
            ly = Yv - y0i.astype(jnp.float32)
            lx = Xv - x0i.astype(jnp.float32)
            hy = 1.0 - ly
            hx = 1.0 - lx
            y0c = jnp.minimum(jnp.maximum(y0i, 0), H - 1)
            x0c = jnp.minimum(jnp.maximum(x0i, 0), W - 1)
            y1c = jnp.minimum(y0c + 1, H - 1)
            x1c = jnp.minimum(x0c + 1, W - 1)
            b = j * 64
            idxv[pl.ds(b, 16)] = y0c * W + x0c
            idxv[pl.ds(b + 16, 16)] = y0c * W + x1c
            idxv[pl.ds(b + 32, 16)] = y1c * W + x0c
            idxv[pl.ds(b + 48, 16)] = y1c * W + x1c
            wbuf[0, pl.ds(j * 16, 16)] = hy * hx
            wbuf[1, pl.ds(j * 16, 16)] = hy * lx
            wbuf[2, pl.ds(j * 16, 16)] = ly * hx
            wbuf[3, pl.ds(j * 16, 16)] = ly * lx
            return carry2

        lax.fori_loop(0, NCHUNK, chunk_body, 0)

        pltpu.async_copy(xt_hbm.at[idxv], gbuf, sem).wait()

        zerov = jnp.zeros((16,), jnp.int32)
        lanesC = lanes * C

        def blend_body(j, carry2):
            w00 = wbuf[0, pl.ds(j * 16, 16)]
            w01 = wbuf[1, pl.ds(j * 16, 16)]
            w10 = wbuf[2, pl.ds(j * 16, 16)]
            w11 = wbuf[3, pl.ds(j * 16, 16)]
            base = j * (64 * C)
            g00 = lanesC + base
            g01 = g00 + 16 * C
            g10 = g00 + 32 * C
            g11 = g00 + 48 * C
            obase = j * 16

            def chan_body(co, carry3):
                c0 = co * 8

                def one(c):
                    v00 = plsc.load_gather(gbuf, [zerov, g00 + c])
                    v01 = plsc.load_gather(gbuf, [zerov, g01 + c])
                    v10 = plsc.load_gather(gbuf, [zerov, g10 + c])
                    v11 = plsc.load_gather(gbuf, [zerov, g11 + c])
                    acc = v00 * w00 + v01 * w01 + v10 * w10 + v11 * w11
                    obuf[c, pl.ds(obase, 16)] = acc

                for cc in range(8):
                    one(c0 + cc)
                return carry3

            lax.fori_loop(0, C // 8, chan_body, 0)
            return carry2

        lax.fori_loop(0, NCHUNK, blend_body, 0)

        pltpu.sync_copy(obuf, out_hbm.at[rbase + i])
        return carry

    lax.fori_loop(0, RPW, roi_body, 0)


@jax.jit
def _roialign_sc(xt, rois_p):
    mesh = plsc.VectorSubcoreMesh(core_axis_name="c", subcore_axis_name="s")
    kfn = functools.partial(
        pl.kernel,
        mesh=mesh,
        out_type=jax.ShapeDtypeStruct((RPAD, C, SS), jnp.float32),
        scratch_types=[
            pltpu.VMEM((RPW, 16), jnp.float32),
            pltpu.VMEM((4 * SS,), jnp.int32),
            pltpu.VMEM((4, SS), jnp.float32),
            pltpu.VMEM((4 * SS, C), jnp.float32),
            pltpu.VMEM((C, SS), jnp.float32),
            pltpu.SemaphoreType.DMA,
        ],
        compiler_params=pltpu.CompilerParams(
            needs_layout_passes=False, use_tc_tiling_on_sc=False),
    )(_sc_body)
    return kfn(xt, rois_p)


def kernel(x, rois):
    N, c, h, w = x.shape
    xt = jnp.transpose(x[0], (1, 2, 0)).reshape(H * W, C)
    rois_p = jnp.pad(rois, ((0, RPAD - rois.shape[0]), (0, 11)))
    out = _roialign_sc(xt, rois_p)
    return out[: rois.shape[0]].reshape(rois.shape[0], C, S, S)


# speedup vs baseline: 1.7453x; 1.7453x over previous
import functools

import jax
import jax.numpy as jnp
from jax import lax
from jax.experimental import pallas as pl
from jax.experimental.pallas import tpu as pltpu
from jax.experimental.pallas import tpu_sc as plsc

S = 12
SS = S * S
C = 96
H = 512
W = 512
NW = 32
RPW = 32
RPAD = NW * RPW
NCHUNK = SS // 16
SP = 145


def _splat(vec, lane):
    idx = jnp.full((16, 1), lane, jnp.int32)
    dnums = lax.GatherDimensionNumbers(
        offset_dims=(), collapsed_slice_dims=(0,), start_index_map=(0,))
    return lax.gather(vec, idx, dnums, (1,),
                      mode=lax.GatherScatterMode.PROMISE_IN_BOUNDS)


def _sc_body(xt_hbm, rois_hbm, out_hbm, roisv, idxv, wbuf, gbuf, obuf, sem):
    wid = lax.axis_index("s") * 2 + lax.axis_index("c")
    rbase = wid * RPW
    pltpu.sync_copy(rois_hbm.at[pl.ds(rbase, RPW)], roisv)

    lanes = lax.iota(jnp.int32, 16)
    chan_idx = [lanes + cc * 16 for cc in range(C // 16)]

    def roi_body(i, carry):
        row = roisv[i, :]
        x1v = _splat(row, 1)
        y1v = _splat(row, 2)
        x2v = _splat(row, 3)
        y2v = _splat(row, 4)
        bwv = (x2v - x1v) / float(S)
        bhv = (y2v - y1v) / float(S)

        def chunk_body(j, carry2):
            s_i = lanes + j * 16
            sy = lax.div(s_i, S)
            sx = s_i - sy * S
            Yv = y1v + (sy.astype(jnp.float32) + 0.5) * bhv
            Xv = x1v + (sx.astype(jnp.float32) + 0.5) * bwv
            y0i = Yv.astype(jnp.int32)
            x0i = Xv.astype(jnp.int32)
            ly = Yv - y0i.astype(jnp.float32)
            lx = Xv - x0i.astype(jnp.float32)
            hy = 1.0 - ly
            hx = 1.0 - lx
            y0c = jnp.minimum(jnp.maximum(y0i, 0), H - 1)
            x0c = jnp.minimum(jnp.maximum(x0i, 0), W - 1)
            y1c = jnp.minimum(y0c + 1, H - 1)
            x1c = jnp.minimum(x0c + 1, W - 1)
            b = j * 64
            idxv[pl.ds(b, 16)] = y0c * W + x0c
            idxv[pl.ds(b + 16, 16)] = y0c * W + x1c
            idxv[pl.ds(b + 32, 16)] = y1c * W + x0c
            idxv[pl.ds(b + 48, 16)] = y1c * W + x1c
            wbuf[0, pl.ds(j * 16, 16)] = hy * hx
            wbuf[1, pl.ds(j * 16, 16)] = hy * lx
            wbuf[2, pl.ds(j * 16, 16)] = ly * hx
            wbuf[3, pl.ds(j * 16, 16)] = ly * lx
            return carry2

        lax.fori_loop(0, NCHUNK, chunk_body, 0)

        pltpu.async_copy(xt_hbm.at[idxv], gbuf, sem).wait()

        def blend_body(jc, carry2):
            w00c = wbuf[0, pl.ds(jc * 16, 16)]
            w01c = wbuf[1, pl.ds(jc * 16, 16)]
            w10c = wbuf[2, pl.ds(jc * 16, 16)]
            w11c = wbuf[3, pl.ds(jc * 16, 16)]
            jc64 = jc * 64
            jc16 = jc * 16

            def samp_body(k, carry3):
                w00s = _splat(w00c, k)
                w01s = _splat(w01c, k)
                w10s = _splat(w10c, k)
                w11s = _splat(w11c, k)
                r0 = jc64 + k
                sv = jnp.broadcast_to(jc16 + k, (16,))
                for cc in range(C // 16):
                    v00 = gbuf[r0, pl.ds(cc * 16, 16)]
                    v01 = gbuf[r0 + 16, pl.ds(cc * 16, 16)]
                    v10 = gbuf[r0 + 32, pl.ds(cc * 16, 16)]
                    v11 = gbuf[r0 + 48, pl.ds(cc * 16, 16)]
                    acc = v00 * w00s + v01 * w01s + v10 * w10s + v11 * w11s
                    plsc.store_scatter(obuf, [chan_idx[cc], sv], acc)
                return carry3

            lax.fori_loop(0, 16, samp_body, 0)
            return carry2

        lax.fori_loop(0, NCHUNK, blend_body, 0)

        pltpu.sync_copy(obuf.at[:, pl.ds(0, SS)], out_hbm.at[rbase + i])
        return carry

    lax.fori_loop(0, RPW, roi_body, 0)


@jax.jit
def _roialign_sc(xt, rois_p):
    mesh = plsc.VectorSubcoreMesh(core_axis_name="c", subcore_axis_name="s")
    kfn = functools.partial(
        pl.kernel,
        mesh=mesh,
        out_type=jax.ShapeDtypeStruct((RPAD, C, SS), jnp.float32),
        scratch_types=[
            pltpu.VMEM((RPW, 16), jnp.float32),
            pltpu.VMEM((4 * SS,), jnp.int32),
            pltpu.VMEM((4, SS), jnp.float32),
            pltpu.VMEM((4 * SS, C), jnp.float32),
            pltpu.VMEM((C, SP), jnp.float32),
            pltpu.SemaphoreType.DMA,
        ],
        compiler_params=pltpu.CompilerParams(
            needs_layout_passes=False, use_tc_tiling_on_sc=False),
    )(_sc_body)
    return kfn(xt, rois_p)


def kernel(x, rois):
    N, c, h, w = x.shape
    xt = jnp.transpose(x[0], (1, 2, 0)).reshape(H * W, C)
    rois_p = jnp.pad(rois, ((0, RPAD - rois.shape[0]), (0, 11)))
    out = _roialign_sc(xt, rois_p)
    return out[: rois.shape[0]].reshape(rois.shape[0], C, S, S)


# speedup vs baseline: 2.1203x; 1.2149x over previous
import functools

import jax
import jax.numpy as jnp
from jax import lax
from jax.experimental import pallas as pl
from jax.experimental.pallas import tpu as pltpu
from jax.experimental.pallas import tpu_sc as plsc

S = 12
SS = S * S
C = 96
H = 512
W = 512
NW = 32
RPW = 32
RPAD = NW * RPW
R = 1000
NCHUNK = SS // 16
SP = 145


def _splat(vec, lane):
    idx = jnp.full((16, 1), lane, jnp.int32)
    dnums = lax.GatherDimensionNumbers(
        offset_dims=(), collapsed_slice_dims=(0,), start_index_map=(0,))
    return lax.gather(vec, idx, dnums, (1,),
                      mode=lax.GatherScatterMode.PROMISE_IN_BOUNDS)


def _sc_body(xt_hbm, rois_hbm, out_hbm, roisv, idx0, idx1, wb0, wb1,
             gb0, gb1, obuf, sem0, sem1):
    wid = lax.axis_index("s") * 2 + lax.axis_index("c")
    rbase = wid * RPW
    pltpu.sync_copy(rois_hbm.at[pl.ds(rbase, RPW)], roisv)

    lanes = lax.iota(jnp.int32, 16)
    chan_idx = [lanes + cc * 16 for cc in range(C // 16)]

    def phase1(i, idxv, wbuf):
        row = roisv[jnp.minimum(i, RPW - 1), :]
        x1v = _splat(row, 1)
        y1v = _splat(row, 2)
        x2v = _splat(row, 3)
        y2v = _splat(row, 4)
        bwv = (x2v - x1v) / float(S)
        bhv = (y2v - y1v) / float(S)

        def chunk_body(j, carry2):
            s_i = lanes + j * 16
            sy = lax.div(s_i, S)
            sx = s_i - sy * S
            Yv = y1v + (sy.astype(jnp.float32) + 0.5) * bhv
            Xv = x1v + (sx.astype(jnp.float32) + 0.5) * bwv
            y0i = Yv.astype(jnp.int32)
            x0i = Xv.astype(jnp.int32)
            ly = Yv - y0i.astype(jnp.float32)
            lx = Xv - x0i.astype(jnp.float32)
            hy = 1.0 - ly
            hx = 1.0 - lx
            y0c = jnp.minimum(jnp.maximum(y0i, 0), H - 1)
            x0c = jnp.minimum(jnp.maximum(x0i, 0), W - 1)
            y1c = jnp.minimum(y0c + 1, H - 1)
            x1c = jnp.minimum(x0c + 1, W - 1)
            b = j * 64
            idxv[pl.ds(b, 16)] = y0c * W + x0c
            idxv[pl.ds(b + 16, 16)] = y0c * W + x1c
            idxv[pl.ds(b + 32, 16)] = y1c * W + x0c
            idxv[pl.ds(b + 48, 16)] = y1c * W + x1c
            wbuf[0, pl.ds(j * 16, 16)] = hy * hx
            wbuf[1, pl.ds(j * 16, 16)] = hy * lx
            wbuf[2, pl.ds(j * 16, 16)] = ly * hx
            wbuf[3, pl.ds(j * 16, 16)] = ly * lx
            return carry2

        lax.fori_loop(0, NCHUNK, chunk_body, 0)

    def blend_out(i, wbuf, gbuf):
        def blend_body(jc, carry2):
            w00c = wbuf[0, pl.ds(jc * 16, 16)]
            w01c = wbuf[1, pl.ds(jc * 16, 16)]
            w10c = wbuf[2, pl.ds(jc * 16, 16)]
            w11c = wbuf[3, pl.ds(jc * 16, 16)]
            jc64 = jc * 64
            jc16 = jc * 16

            def samp_body(k, carry3):
                w00s = _splat(w00c, k)
                w01s = _splat(w01c, k)
                w10s = _splat(w10c, k)
                w11s = _splat(w11c, k)
                r0 = jc64 + k
                sv = jnp.broadcast_to(jc16 + k, (16,))
                for cc in range(C // 16):
                    v00 = gbuf[r0, pl.ds(cc * 16, 16)]
                    v01 = gbuf[r0 + 16, pl.ds(cc * 16, 16)]
                    v10 = gbuf[r0 + 32, pl.ds(cc * 16, 16)]
                    v11 = gbuf[r0 + 48, pl.ds(cc * 16, 16)]
                    acc = v00 * w00s + v01 * w01s + v10 * w10s + v11 * w11s
                    plsc.store_scatter(obuf, [chan_idx[cc], sv], acc)
                return carry3

            lax.fori_loop(0, 16, samp_body, 0)
            return carry2

        lax.fori_loop(0, NCHUNK, blend_body, 0)

        ridx = rbase + i

        @pl.when(ridx < R)
        def _():
            pltpu.sync_copy(obuf.at[:, pl.ds(0, SS)], out_hbm.at[ridx])

    phase1(0, idx0, wb0)
    pltpu.async_copy(xt_hbm.at[idx0], gb0, sem0)

    def pair_body(p, carry):
        a = 2 * p
        phase1(a + 1, idx1, wb1)
        pltpu.async_copy(xt_hbm.at[idx1], gb1, sem1)
        pltpu.make_async_copy(xt_hbm.at[idx0], gb0, sem0).wait()
        blend_out(a, wb0, gb0)
        phase1(a + 2, idx0, wb0)

        @pl.when(p < RPW // 2 - 1)
        def _():
            pltpu.async_copy(xt_hbm.at[idx0], gb0, sem0)

        pltpu.make_async_copy(xt_hbm.at[idx1], gb1, sem1).wait()
        blend_out(a + 1, wb1, gb1)
        return carry

    lax.fori_loop(0, RPW // 2, pair_body, 0)


def _tr_body(x_ref, o_ref):
    o_ref[...] = x_ref[...].T


@jax.jit
def _roialign_sc(x2d, rois_p):
    xt = pl.pallas_call(
        _tr_body,
        grid=(64,),
        in_specs=[pl.BlockSpec((C, (H * W) // 64), lambda i: (0, i))],
        out_specs=pl.BlockSpec(((H * W) // 64, C), lambda i: (i, 0)),
        out_shape=jax.ShapeDtypeStruct((H * W, C), jnp.float32),
    )(x2d)

    mesh = plsc.VectorSubcoreMesh(core_axis_name="c", subcore_axis_name="s")
    kfn = functools.partial(
        pl.kernel,
        mesh=mesh,
        out_type=jax.ShapeDtypeStruct((R, C, SS), jnp.float32),
        scratch_types=[
            pltpu.VMEM((RPW, 16), jnp.float32),
            pltpu.VMEM((4 * SS,), jnp.int32),
            pltpu.VMEM((4 * SS,), jnp.int32),
            pltpu.VMEM((4, SS), jnp.float32),
            pltpu.VMEM((4, SS), jnp.float32),
            pltpu.VMEM((4 * SS, C), jnp.float32),
            pltpu.VMEM((4 * SS, C), jnp.float32),
            pltpu.VMEM((C, SP), jnp.float32),
            pltpu.SemaphoreType.DMA,
            pltpu.SemaphoreType.DMA,
        ],
        compiler_params=pltpu.CompilerParams(
            needs_layout_passes=False, use_tc_tiling_on_sc=False),
    )(_sc_body)
    return kfn(xt, rois_p)


def kernel(x, rois):
    x2d = x[0].reshape(C, H * W)
    rois_p = jnp.pad(rois, ((0, RPAD - rois.shape[0]), (0, 11)))
    out = _roialign_sc(x2d, rois_p)
    return out.reshape(R, C, S, S)
